# baseline (speedup 1.0000x reference)
import functools

import jax
import jax.numpy as jnp
from jax import lax
from jax.experimental import pallas as pl
from jax.experimental.pallas import tpu as pltpu
from jax.experimental.pallas import tpu_sc as plsc

_VOCAB = 1000000
_E = 64
_B = 4096
_L = 200
_LANES = 16
_NC = 2
_NS = 16
_NW = _NC * _NS

_W = 384
_NWIN = _VOCAB // _W
_TAIL = _VOCAB - _NWIN * _W
_KMAX = -(-_NWIN // _NW)


def _tr_body(tT_hbm, out_hbm, in_v, out_v, tin_v, tout_v, sem_i, sem_o):
  wid = lax.axis_index("s") * _NC + lax.axis_index("c")

  def start_in(widx, buf):
    pltpu.async_copy(tT_hbm.at[:, pl.ds(widx * _W, _W)],
                     in_v.at[buf], sem_i[buf])

  def transpose_window(buf):
    ci = lax.iota(jnp.int32, _LANES)

    def body(q, carry):
      for h in range(2):
        rv = jnp.full((_LANES,), 0, jnp.int32) + (2 * q + h)
        for c0 in range(0, _E, _LANES):
          v = plsc.load_gather(in_v.at[buf], [ci + c0, rv])
          out_v[buf, q, pl.ds(h * _E + c0, _LANES)] = v
      return carry

    lax.fori_loop(0, _W // 2, body, 0)

  def start_out(widx, buf):
    pltpu.async_copy(out_v.at[buf],
                     out_hbm.at[pl.ds(widx * (_W // 2), _W // 2)], sem_o[buf])

  @pl.when(wid < _NWIN)
  def _():
    start_in(wid, 0)

  def step(k, buf):
    widx = wid + _NW * k

    @pl.when(widx < _NWIN)
    def _():
      @pl.when(widx + _NW < _NWIN)
      def _():
        start_in(widx + _NW, 1 - buf)

      pltpu.make_async_copy(
          tT_hbm.at[:, pl.ds(0, _W)], in_v.at[buf], sem_i[buf]).wait()

      @pl.when(k >= 2)
      def _():
        pltpu.make_async_copy(
            out_v.at[buf], out_hbm.at[pl.ds(0, _W // 2)], sem_o[buf]).wait()

      transpose_window(buf)
      start_out(widx, buf)

  def outer(i, carry):
    for j in range(2):
      step(2 * i + j, j)
    return carry

  lax.fori_loop(0, _KMAX // 2, outer, 0)

  for j in range(2):
    pltpu.make_async_copy(
        out_v.at[j], out_hbm.at[pl.ds(0, _W // 2)], sem_o[j]).wait()

  @pl.when(wid == 0)
  def _():
    pltpu.sync_copy(tT_hbm.at[:, pl.ds(_NWIN * _W, _TAIL)], tin_v)
    ci = lax.iota(jnp.int32, _LANES)

    def tbody(q, carry):
      for h in range(2):
        rv = jnp.full((_LANES,), 0, jnp.int32) + (2 * q + h)
        for c0 in range(0, _E, _LANES):
          v = plsc.load_gather(tin_v, [ci + c0, rv])
          tout_v[q, pl.ds(h * _E + c0, _LANES)] = v
      return carry

    lax.fori_loop(0, _TAIL // 2, tbody, 0)
    pltpu.sync_copy(tout_v,
                    out_hbm.at[pl.ds(_NWIN * _W // 2, _TAIL // 2)])


_tr = functools.partial(
    pl.kernel,
    mesh=plsc.VectorSubcoreMesh(core_axis_name="c", subcore_axis_name="s"),
    out_type=jax.ShapeDtypeStruct((_VOCAB // 2, 2 * _E), jnp.float32),
    scratch_types=[
        pltpu.VMEM((2, _E, _W), jnp.float32),
        pltpu.VMEM((2, _W // 2, 2 * _E), jnp.float32),
        pltpu.VMEM((_E, _TAIL), jnp.float32),
        pltpu.VMEM((_TAIL // 2, 2 * _E), jnp.float32),
        [pltpu.SemaphoreType.DMA] * 2,
        [pltpu.SemaphoreType.DMA] * 2,
    ],
    compiler_params=pltpu.CompilerParams(
        use_tc_tiling_on_sc=True, needs_layout_passes=False),
)(_tr_body)

_BPW = _B // _NW
_NBUF = 4
_CHUNKS = ((0, 128), (128, 72))


def _bow_body(data_hbm, len_hbm, table_hbm, out_hbm,
              idx_v, len_v, rows_v, outb_v, sem_g, sem_o):
  wid = lax.axis_index("s") * _NC + lax.axis_index("c")
  base = wid * _BPW

  pltpu.sync_copy(data_hbm.at[pl.ds(base * _L, _BPW * _L)],
                  idx_v.at[pl.ds(0, _BPW * _L)])
  pltpu.sync_copy(len_hbm.at[pl.ds(base, _BPW)], len_v.at[pl.ds(0, _BPW)])

  def start_gathers(b, buf):
    for off, n in _CHUNKS:
      pltpu.async_copy(
          table_hbm.at[idx_v.at[pl.ds(b * _L + off, n)]],
          rows_v.at[buf, pl.ds(off, n)],
          sem_g[buf])

  def wait_gathers(buf):
    pltpu.make_async_copy(
        table_hbm.at[pl.ds(0, _L)], rows_v.at[buf], sem_g[buf]).wait()

  def compute(b, buf):
    def fold(i, accs):
      l = 2 * i
      out = []
      for c in range(4):
        s = pl.ds(c * _LANES, _LANES)
        out.append(accs[c] + rows_v[buf, l, s])
        out.append(accs[c + 4] + rows_v[buf, l + 1, s])
      return (out[0], out[2], out[4], out[6], out[1], out[3], out[5], out[7])

    zero = jnp.zeros((_LANES,), jnp.float32)
    accs = lax.fori_loop(0, _L // 2, fold, (zero,) * 8, unroll=5)

    lenf = len_v[pl.ds(b, _LANES)][0].astype(jnp.float32)
    recip = jnp.full((_LANES,), 1.0, jnp.float32) / lenf
    for c in range(4):
      outb_v[buf, 0, pl.ds(c * _LANES, _LANES)] = (accs[c] + accs[c + 4]) * recip

  for j in range(_NBUF - 1):
    start_gathers(j, j)

  def step(b, buf):
    wait_gathers(buf)

    @pl.when(b >= _NBUF)
    def _():
      pltpu.make_async_copy(
          outb_v.at[buf], out_hbm.at[pl.ds(0, 1)], sem_o[buf]).wait()

    compute(b, buf)
    pltpu.async_copy(
        outb_v.at[buf], out_hbm.at[pl.ds(base + b, 1)], sem_o[buf])

    @pl.when(b + _NBUF - 1 < _BPW)
    def _():
      start_gathers(b + _NBUF - 1, (buf + _NBUF - 1) % _NBUF)

  def outer(i, carry):
    for j in range(_NBUF):
      step(_NBUF * i + j, j)
    return carry

  lax.fori_loop(0, _BPW // _NBUF, outer, 0)

  for j in range(_NBUF):
    pltpu.make_async_copy(
        outb_v.at[j], out_hbm.at[pl.ds(0, 1)], sem_o[j]).wait()


_bow = functools.partial(
    pl.kernel,
    mesh=plsc.VectorSubcoreMesh(core_axis_name="c", subcore_axis_name="s"),
    out_type=jax.ShapeDtypeStruct((_B, _E), jnp.float32),
    scratch_types=[
        pltpu.VMEM((_BPW * _L,), jnp.int32),
        pltpu.VMEM((_BPW + _LANES,), jnp.int32),
        pltpu.VMEM((_NBUF, _L, _E), jnp.float32),
        pltpu.VMEM((_NBUF, 1, _E), jnp.float32),
        [pltpu.SemaphoreType.DMA] * _NBUF,
        [pltpu.SemaphoreType.DMA] * _NBUF,
    ],
    compiler_params=pltpu.CompilerParams(use_tc_tiling_on_sc=False),
)(_bow_body)


@jax.jit
def kernel(data_bl, length_b, table):
  data_flat = data_bl.reshape(_B * _L)
  len_flat = length_b.reshape(_B)
  t128 = _tr(table.T)
  t_lin = t128.reshape(_VOCAB, _E)
  return _bow(data_flat, len_flat, t_lin)


# speedup vs baseline: 3.6141x; 3.6141x over previous
import functools

import jax
import jax.numpy as jnp
from jax import lax
from jax.experimental import pallas as pl
from jax.experimental.pallas import tpu as pltpu
from jax.experimental.pallas import tpu_sc as plsc

_VOCAB = 1000000
_E = 64
_B = 4096
_L = 200
_LANES = 16
_NC = 2
_NS = 16
_NW = _NC * _NS

_BPW = _B // _NW
_NBUF = 4
_CHUNKS = ((0, 128), (128, 72))


def _bow_body(data_hbm, len_hbm, table_hbm, out_hbm,
              idx_v, len_v, rows_v, outb_v, sem_g, sem_o):
  wid = lax.axis_index("s") * _NC + lax.axis_index("c")
  base = wid * _BPW

  pltpu.sync_copy(data_hbm.at[pl.ds(base * _L, _BPW * _L)],
                  idx_v.at[pl.ds(0, _BPW * _L)])
  pltpu.sync_copy(len_hbm.at[pl.ds(base, _BPW)], len_v.at[pl.ds(0, _BPW)])

  def start_gathers(b, buf):
    for off, n in _CHUNKS:
      pltpu.async_copy(
          table_hbm.at[idx_v.at[pl.ds(b * _L + off, n)]],
          rows_v.at[buf, pl.ds(off, n)],
          sem_g[buf])

  def wait_gathers(buf):
    pltpu.make_async_copy(
        table_hbm.at[pl.ds(0, _L)], rows_v.at[buf], sem_g[buf]).wait()

  def compute(b, buf):
    def fold(i, accs):
      l = 2 * i
      out = []
      for c in range(4):
        s = pl.ds(c * _LANES, _LANES)
        out.append(accs[c] + rows_v[buf, l, s])
        out.append(accs[c + 4] + rows_v[buf, l + 1, s])
      return (out[0], out[2], out[4], out[6], out[1], out[3], out[5], out[7])

    zero = jnp.zeros((_LANES,), jnp.float32)
    accs = lax.fori_loop(0, _L // 2, fold, (zero,) * 8, unroll=5)

    lenf = len_v[pl.ds(b, _LANES)][0].astype(jnp.float32)
    recip = jnp.full((_LANES,), 1.0, jnp.float32) / lenf
    for c in range(4):
      outb_v[buf, 0, pl.ds(c * _LANES, _LANES)] = (accs[c] + accs[c + 4]) * recip

  for j in range(_NBUF - 1):
    start_gathers(j, j)

  def step(b, buf):
    wait_gathers(buf)

    @pl.when(b >= _NBUF)
    def _():
      pltpu.make_async_copy(
          outb_v.at[buf], out_hbm.at[pl.ds(0, 1)], sem_o[buf]).wait()

    compute(b, buf)
    pltpu.async_copy(
        outb_v.at[buf], out_hbm.at[pl.ds(base + b, 1)], sem_o[buf])

    @pl.when(b + _NBUF - 1 < _BPW)
    def _():
      start_gathers(b + _NBUF - 1, (buf + _NBUF - 1) % _NBUF)

  def outer(i, carry):
    for j in range(_NBUF):
      step(_NBUF * i + j, j)
    return carry

  lax.fori_loop(0, _BPW // _NBUF, outer, 0)

  for j in range(_NBUF):
    pltpu.make_async_copy(
        outb_v.at[j], out_hbm.at[pl.ds(0, 1)], sem_o[j]).wait()


_bow = functools.partial(
    pl.kernel,
    mesh=plsc.VectorSubcoreMesh(core_axis_name="c", subcore_axis_name="s"),
    out_type=jax.ShapeDtypeStruct((_B, _E), jnp.float32),
    scratch_types=[
        pltpu.VMEM((_BPW * _L,), jnp.int32),
        pltpu.VMEM((_BPW + _LANES,), jnp.int32),
        pltpu.VMEM((_NBUF, _L, 2 * _E), jnp.float32),
        pltpu.VMEM((_NBUF, 1, _E), jnp.float32),
        [pltpu.SemaphoreType.DMA] * _NBUF,
        [pltpu.SemaphoreType.DMA] * _NBUF,
    ],
    compiler_params=pltpu.CompilerParams(use_tc_tiling_on_sc=False),
)(_bow_body)


_TCW = 16384
_TCG = -(-_VOCAB // _TCW)


def _tc_tr_body(in_ref, out_ref):
  out_ref[:, pl.ds(0, _E)] = in_ref[...].T
  out_ref[:, pl.ds(_E, _E)] = jnp.zeros((_TCW, _E), jnp.float32)


_tc_tr = pl.pallas_call(
    _tc_tr_body,
    grid=(_TCG,),
    in_specs=[pl.BlockSpec((_E, _TCW), lambda w: (0, w))],
    out_specs=pl.BlockSpec((_TCW, 2 * _E), lambda w: (w, 0)),
    out_shape=jax.ShapeDtypeStruct((_VOCAB, 2 * _E), jnp.float32),
)


@jax.jit
def kernel(data_bl, length_b, table):
  data_flat = data_bl.reshape(_B * _L)
  len_flat = length_b.reshape(_B)
  t128 = _tc_tr(table.T)
  return _bow(data_flat, len_flat, t128)


# speedup vs baseline: 4.8840x; 1.3514x over previous
import functools

import jax
import jax.numpy as jnp
from jax import lax
from jax.experimental import pallas as pl
from jax.experimental.pallas import tpu as pltpu
from jax.experimental.pallas import tpu_sc as plsc

_VOCAB = 1000000
_E = 64
_B = 4096
_L = 200
_LANES = 16
_NC = 2
_NS = 16
_NW = _NC * _NS

_BPW = _B // _NW
_NBUF = 8
_CHUNKS = ((0, 128), (128, 72))


def _bow_body(data_hbm, len_hbm, table_hbm, out_hbm,
              idx_v, len_v, rows_v, outb_v, sem_g, sem_o):
  wid = lax.axis_index("s") * _NC + lax.axis_index("c")
  base = wid * _BPW

  pltpu.sync_copy(data_hbm.at[pl.ds(base * _L, _BPW * _L)],
                  idx_v.at[pl.ds(0, _BPW * _L)])
  pltpu.sync_copy(len_hbm.at[pl.ds(base, _BPW)], len_v.at[pl.ds(0, _BPW)])

  def remap(k, carry):
    v = idx_v[pl.ds(k * _LANES, _LANES)]
    j = (lax.shift_left(lax.shift_right_logical(v, 14), 14)
         + lax.shift_left(v & 8191, 1)
         + (lax.shift_right_logical(v, 13) & 1))
    idx_v[pl.ds(k * _LANES, _LANES)] = j
    return carry

  lax.fori_loop(0, _BPW * _L // _LANES, remap, 0, unroll=4)

  def start_gathers(b, buf):
    for off, n in _CHUNKS:
      pltpu.async_copy(
          table_hbm.at[idx_v.at[pl.ds(b * _L + off, n)]],
          rows_v.at[buf, pl.ds(off, n)],
          sem_g[buf])

  def wait_gathers(buf):
    pltpu.make_async_copy(
        table_hbm.at[pl.ds(0, _L)], rows_v.at[buf], sem_g[buf]).wait()

  def compute(b, buf):
    def fold(i, accs):
      l = 2 * i
      out = []
      for c in range(4):
        s = pl.ds(c * _LANES, _LANES)
        out.append(accs[c] + rows_v[buf, l, s])
        out.append(accs[c + 4] + rows_v[buf, l + 1, s])
      return (out[0], out[2], out[4], out[6], out[1], out[3], out[5], out[7])

    zero = jnp.zeros((_LANES,), jnp.float32)
    accs = lax.fori_loop(0, _L // 2, fold, (zero,) * 8, unroll=5)

    lenf = len_v[pl.ds(b, _LANES)][0].astype(jnp.float32)
    recip = jnp.full((_LANES,), 1.0, jnp.float32) / lenf
    for c in range(4):
      outb_v[buf, 0, pl.ds(c * _LANES, _LANES)] = (accs[c] + accs[c + 4]) * recip

  for j in range(_NBUF - 1):
    start_gathers(j, j)

  def step(b, buf):
    wait_gathers(buf)

    @pl.when(b >= _NBUF)
    def _():
      pltpu.make_async_copy(
          outb_v.at[buf], out_hbm.at[pl.ds(0, 1)], sem_o[buf]).wait()

    compute(b, buf)
    pltpu.async_copy(
        outb_v.at[buf], out_hbm.at[pl.ds(base + b, 1)], sem_o[buf])

    @pl.when(b + _NBUF - 1 < _BPW)
    def _():
      start_gathers(b + _NBUF - 1, (buf + _NBUF - 1) % _NBUF)

  def outer(i, carry):
    for j in range(_NBUF):
      step(_NBUF * i + j, j)
    return carry

  lax.fori_loop(0, _BPW // _NBUF, outer, 0)

  for j in range(_NBUF):
    pltpu.make_async_copy(
        outb_v.at[j], out_hbm.at[pl.ds(0, 1)], sem_o[j]).wait()


_bow = functools.partial(
    pl.kernel,
    mesh=plsc.VectorSubcoreMesh(core_axis_name="c", subcore_axis_name="s"),
    out_type=jax.ShapeDtypeStruct((_B, _E), jnp.float32),
    scratch_types=[
        pltpu.VMEM((_BPW * _L,), jnp.int32),
        pltpu.VMEM((_BPW + _LANES,), jnp.int32),
        pltpu.VMEM((_NBUF, _L, _E), jnp.float32),
        pltpu.VMEM((_NBUF, 1, _E), jnp.float32),
        [pltpu.SemaphoreType.DMA] * _NBUF,
        [pltpu.SemaphoreType.DMA] * _NBUF,
    ],
    compiler_params=pltpu.CompilerParams(use_tc_tiling_on_sc=False),
)(_bow_body)


_TCW = 16384
_TCG = -(-_VOCAB // _TCW)
_HALF = _TCW // 2
_N2 = _TCG * _TCW


def _tc_tr_body(in_ref, out_ref):
  out_ref[:, pl.ds(0, _E)] = in_ref[:, pl.ds(0, _HALF)].T
  out_ref[:, pl.ds(_E, _E)] = in_ref[:, pl.ds(_HALF, _HALF)].T


_tc_tr = pl.pallas_call(
    _tc_tr_body,
    grid=(_TCG,),
    in_specs=[pl.BlockSpec((_E, _TCW), lambda w: (0, w))],
    out_specs=pl.BlockSpec((_HALF, 2 * _E), lambda w: (w, 0)),
    out_shape=jax.ShapeDtypeStruct((_TCG * _HALF, 2 * _E), jnp.float32),
)


@jax.jit
def kernel(data_bl, length_b, table):
  data_flat = data_bl.reshape(_B * _L)
  len_flat = length_b.reshape(_B)
  t128 = _tc_tr(table.T)
  t2 = t128.reshape(_N2, _E)
  return _bow(data_flat, len_flat, t2)


# speedup vs baseline: 5.1194x; 1.0482x over previous
import functools

import jax
import jax.numpy as jnp
from jax import lax
from jax.experimental import pallas as pl
from jax.experimental.pallas import tpu as pltpu
from jax.experimental.pallas import tpu_sc as plsc

_VOCAB = 1000000
_E = 64
_B = 4096
_L = 200
_LANES = 16
_NC = 2
_NS = 16
_NW = _NC * _NS

_BPW = _B // _NW
_NBUF = 8
_SH = 15
_HALFM = 1 << (_SH - 1)
_CHUNKS = ((0, 128), (128, 72))


def _bow_body(data_hbm, len_hbm, table_hbm, out_hbm,
              idx_v, len_v, rows_v, outb_v, sem_g, sem_o):
  wid = lax.axis_index("s") * _NC + lax.axis_index("c")
  base = wid * _BPW

  pltpu.sync_copy(data_hbm.at[pl.ds(base * _L, _BPW * _L)],
                  idx_v.at[pl.ds(0, _BPW * _L)])
  pltpu.sync_copy(len_hbm.at[pl.ds(base, _BPW)], len_v.at[pl.ds(0, _BPW)])

  def remap(k, carry):
    v = idx_v[pl.ds(k * _LANES, _LANES)]
    j = (lax.shift_left(lax.shift_right_logical(v, _SH), _SH)
         + lax.shift_left(v & (_HALFM - 1), 1)
         + (lax.shift_right_logical(v, _SH - 1) & 1))
    idx_v[pl.ds(k * _LANES, _LANES)] = j
    return carry

  lax.fori_loop(0, _BPW * _L // _LANES, remap, 0, unroll=4)

  def start_gathers(b, buf):
    for off, n in _CHUNKS:
      pltpu.async_copy(
          table_hbm.at[idx_v.at[pl.ds(b * _L + off, n)]],
          rows_v.at[buf, pl.ds(off, n)],
          sem_g[buf])

  def wait_gathers(buf):
    pltpu.make_async_copy(
        table_hbm.at[pl.ds(0, _L)], rows_v.at[buf], sem_g[buf]).wait()

  def compute(b, buf):
    def fold(i, accs):
      l = 2 * i
      out = []
      for c in range(4):
        s = pl.ds(c * _LANES, _LANES)
        out.append(accs[c] + rows_v[buf, l, s])
        out.append(accs[c + 4] + rows_v[buf, l + 1, s])
      return (out[0], out[2], out[4], out[6], out[1], out[3], out[5], out[7])

    zero = jnp.zeros((_LANES,), jnp.float32)
    accs = lax.fori_loop(0, _L // 2, fold, (zero,) * 8, unroll=5)

    lenf = len_v[pl.ds(b, _LANES)][0].astype(jnp.float32)
    recip = jnp.full((_LANES,), 1.0, jnp.float32) / lenf
    for c in range(4):
      outb_v[buf, 0, pl.ds(c * _LANES, _LANES)] = (accs[c] + accs[c + 4]) * recip

  for j in range(_NBUF - 1):
    start_gathers(j, j)

  def step(b, buf):
    wait_gathers(buf)

    @pl.when(b >= _NBUF)
    def _():
      pltpu.make_async_copy(
          outb_v.at[buf], out_hbm.at[pl.ds(0, 1)], sem_o[buf]).wait()

    compute(b, buf)
    pltpu.async_copy(
        outb_v.at[buf], out_hbm.at[pl.ds(base + b, 1)], sem_o[buf])

    @pl.when(b + _NBUF - 1 < _BPW)
    def _():
      start_gathers(b + _NBUF - 1, (buf + _NBUF - 1) % _NBUF)

  def outer(i, carry):
    for j in range(_NBUF):
      step(_NBUF * i + j, j)
    return carry

  lax.fori_loop(0, _BPW // _NBUF, outer, 0)

  for j in range(_NBUF):
    pltpu.make_async_copy(
        outb_v.at[j], out_hbm.at[pl.ds(0, 1)], sem_o[j]).wait()


_bow = functools.partial(
    pl.kernel,
    mesh=plsc.VectorSubcoreMesh(core_axis_name="c", subcore_axis_name="s"),
    out_type=jax.ShapeDtypeStruct((_B, _E), jnp.float32),
    scratch_types=[
        pltpu.VMEM((_BPW * _L,), jnp.int32),
        pltpu.VMEM((_BPW + _LANES,), jnp.int32),
        pltpu.VMEM((_NBUF, _L, _E), jnp.float32),
        pltpu.VMEM((_NBUF, 1, _E), jnp.float32),
        [pltpu.SemaphoreType.DMA] * _NBUF,
        [pltpu.SemaphoreType.DMA] * _NBUF,
    ],
    compiler_params=pltpu.CompilerParams(use_tc_tiling_on_sc=False),
)(_bow_body)


_TCW = 32768
_TCG = -(-_VOCAB // _TCW)
_HALF = _TCW // 2
_N2 = _TCG * _TCW


def _tc_tr_body(in_ref, out_ref):
  out_ref[:, pl.ds(0, _E)] = in_ref[:, pl.ds(0, _HALF)].T
  out_ref[:, pl.ds(_E, _E)] = in_ref[:, pl.ds(_HALF, _HALF)].T


_tc_tr = pl.pallas_call(
    _tc_tr_body,
    grid=(_TCG,),
    in_specs=[pl.BlockSpec((_E, _TCW), lambda w: (0, w))],
    out_specs=pl.BlockSpec((_HALF, 2 * _E), lambda w: (w, 0)),
    out_shape=jax.ShapeDtypeStruct((_TCG * _HALF, 2 * _E), jnp.float32),
)


@jax.jit
def kernel(data_bl, length_b, table):
  data_flat = data_bl.reshape(_B * _L)
  len_flat = length_b.reshape(_B)
  t128 = _tc_tr(table.T)
  t2 = t128.reshape(_N2, _E)
  return _bow(data_flat, len_flat, t2)


# speedup vs baseline: 5.1244x; 1.0010x over previous
import functools

import jax
import jax.numpy as jnp
from jax import lax
from jax.experimental import pallas as pl
from jax.experimental.pallas import tpu as pltpu
from jax.experimental.pallas import tpu_sc as plsc

_VOCAB = 1000000
_E = 64
_B = 4096
_L = 200
_LANES = 16
_NC = 2
_NS = 16
_NW = _NC * _NS

_BPW = _B // _NW
_NBUF = 8
_SH = 15
_HALFM = 1 << (_SH - 1)
_CHUNKS = ((0, 128), (128, 72))


def _bow_body(data_hbm, len_hbm, table_hbm, out_hbm,
              idx_v, len_v, rows_v, outb_v, sem_g, sem_o):
  wid = lax.axis_index("s") * _NC + lax.axis_index("c")
  base = wid * _BPW

  pltpu.sync_copy(data_hbm.at[pl.ds(base * _L, _BPW * _L)],
                  idx_v.at[pl.ds(0, _BPW * _L)])
  pltpu.sync_copy(len_hbm.at[pl.ds(base, _BPW)], len_v.at[pl.ds(0, _BPW)])

  def remap(k, carry):
    v = idx_v[pl.ds(k * _LANES, _LANES)]
    j = (lax.shift_left(lax.shift_right_logical(v, _SH), _SH)
         + lax.shift_left(v & (_HALFM - 1), 1)
         + (lax.shift_right_logical(v, _SH - 1) & 1))
    idx_v[pl.ds(k * _LANES, _LANES)] = j
    return carry

  lax.fori_loop(0, _BPW * _L // _LANES, remap, 0, unroll=4)

  def start_gathers(b, buf):
    for off, n in _CHUNKS:
      pltpu.async_copy(
          table_hbm.at[idx_v.at[pl.ds(b * _L + off, n)]],
          rows_v.at[buf, pl.ds(off, n)],
          sem_g[buf])

  def wait_gathers(buf):
    pltpu.make_async_copy(
        table_hbm.at[pl.ds(0, _L)], rows_v.at[buf], sem_g[buf]).wait()

  def compute(b, buf):
    def fold(i, accs):
      l = 2 * i
      out = []
      for c in range(4):
        s = pl.ds(c * _LANES, _LANES)
        out.append(accs[c] + rows_v[buf, l, s])
        out.append(accs[c + 4] + rows_v[buf, l + 1, s])
      return (out[0], out[2], out[4], out[6], out[1], out[3], out[5], out[7])

    zero = jnp.zeros((_LANES,), jnp.float32)
    accs = lax.fori_loop(0, _L // 2, fold, (zero,) * 8, unroll=5)

    lenf = len_v[pl.ds(b, _LANES)][0].astype(jnp.float32)
    recip = jnp.full((_LANES,), 1.0, jnp.float32) / lenf
    for c in range(4):
      outb_v[buf, 0, pl.ds(c * _LANES, _LANES)] = (accs[c] + accs[c + 4]) * recip

  for j in range(_NBUF - 1):
    start_gathers(j, j)

  def step(b, buf):
    wait_gathers(buf)

    @pl.when(b >= _NBUF)
    def _():
      pltpu.make_async_copy(
          outb_v.at[buf], out_hbm.at[pl.ds(0, 1)], sem_o[buf]).wait()

    compute(b, buf)
    pltpu.async_copy(
        outb_v.at[buf], out_hbm.at[pl.ds(base + b, 1)], sem_o[buf])

    @pl.when(b + _NBUF - 1 < _BPW)
    def _():
      start_gathers(b + _NBUF - 1, (buf + _NBUF - 1) % _NBUF)

  def outer(i, carry):
    for j in range(_NBUF):
      step(_NBUF * i + j, j)
    return carry

  lax.fori_loop(0, _BPW // _NBUF, outer, 0)

  for j in range(_NBUF):
    pltpu.make_async_copy(
        outb_v.at[j], out_hbm.at[pl.ds(0, 1)], sem_o[j]).wait()


_bow = functools.partial(
    pl.kernel,
    mesh=plsc.VectorSubcoreMesh(core_axis_name="c", subcore_axis_name="s"),
    out_type=jax.ShapeDtypeStruct((_B, _E), jnp.float32),
    scratch_types=[
        pltpu.VMEM((_BPW * _L,), jnp.int32),
        pltpu.VMEM((_BPW + _LANES,), jnp.int32),
        pltpu.VMEM((_NBUF, _L, _E), jnp.float32),
        pltpu.VMEM((_NBUF, 1, _E), jnp.float32),
        [pltpu.SemaphoreType.DMA] * _NBUF,
        [pltpu.SemaphoreType.DMA] * _NBUF,
    ],
    compiler_params=pltpu.CompilerParams(use_tc_tiling_on_sc=False),
)(_bow_body)


_TCW = 32768
_TCG = -(-_VOCAB // _TCW)
_HALF = _TCW // 2
_N2 = _TCG * _TCW


def _tc_tr_body(in_ref, out_ref):
  out_ref[...] = jnp.concatenate(
      [in_ref[:, pl.ds(0, _HALF)].T, in_ref[:, pl.ds(_HALF, _HALF)].T],
      axis=1)


_tc_tr = pl.pallas_call(
    _tc_tr_body,
    grid=(_TCG,),
    in_specs=[pl.BlockSpec((_E, _TCW), lambda w: (0, w))],
    out_specs=pl.BlockSpec((_HALF, 2 * _E), lambda w: (w, 0)),
    out_shape=jax.ShapeDtypeStruct((_TCG * _HALF, 2 * _E), jnp.float32),
)


@jax.jit
def kernel(data_bl, length_b, table):
  data_flat = data_bl.reshape(_B * _L)
  len_flat = length_b.reshape(_B)
  t128 = _tc_tr(table.T)
  t2 = t128.reshape(_N2, _E)
  return _bow(data_flat, len_flat, t2)
